# baseline (speedup 1.0000x reference)
import jax
import jax.numpy as jnp
from jax import lax
from jax.experimental import pallas as pl
from jax.experimental.pallas import tpu as pltpu

N_TOKENS = 1024
DIM = 256
N_Q = 4
BINS = 512

BLOCK_T = 256


def _rvq_kernel(h_ref, cb_ref, codes_ref, quant_ref, splits_ref, chalf_ref):
    pid = pl.program_id(0)

    @pl.when(pid == 0)
    def _prep():
        cb_all = cb_ref[:].reshape(N_Q * BINS, DIM)
        ones8 = jnp.ones((8, DIM), jnp.float32)
        cnorm8 = lax.dot_general(
            ones8, cb_all * cb_all,
            dimension_numbers=(((1,), (1,)), ((), ())),
            preferred_element_type=jnp.float32,
            precision=lax.Precision.HIGHEST,
        )
        chalf_ref[:] = 0.5 * cnorm8
        r = cb_all
        for p in range(3):
            part = r.astype(jnp.bfloat16)
            splits_ref[p] = part
            r = r - part.astype(jnp.float32)

    residual = h_ref[:]
    idx_cols = []
    for i in range(N_Q):
        cb = cb_ref[i]
        dots = lax.dot_general(
            residual, cb,
            dimension_numbers=(((1,), (1,)), ((), ())),
            preferred_element_type=jnp.float32,
            precision=lax.Precision.HIGHEST,
        )
        scores = dots - chalf_ref[0:1, i * BINS:(i + 1) * BINS]
        maxs = jnp.max(scores, axis=1, keepdims=True)
        iota = lax.broadcasted_iota(jnp.int32, scores.shape, 1)
        idx2d = jnp.min(jnp.where(scores == maxs, iota, BINS),
                        axis=1, keepdims=True)
        onehot = (iota == idx2d).astype(jnp.bfloat16)
        chosen = jnp.zeros_like(residual)
        for p in range(3):
            chosen = chosen + lax.dot_general(
                onehot, splits_ref[p, i * BINS:(i + 1) * BINS],
                dimension_numbers=(((1,), (0,)), ((), ())),
                preferred_element_type=jnp.float32,
            )
        residual = residual - chosen
        idx_cols.append(idx2d)
    codes_ref[:] = jnp.concatenate(idx_cols, axis=1)
    quant_ref[:] = h_ref[:] - residual


def kernel(hidden_states, codebooks):
    codes_t, quant = pl.pallas_call(
        _rvq_kernel,
        grid=(N_TOKENS // BLOCK_T,),
        in_specs=[
            pl.BlockSpec((BLOCK_T, DIM), lambda j: (j, 0)),
            pl.BlockSpec((N_Q, BINS, DIM), lambda j: (0, 0, 0)),
        ],
        out_specs=[
            pl.BlockSpec((BLOCK_T, N_Q), lambda j: (j, 0)),
            pl.BlockSpec((BLOCK_T, DIM), lambda j: (j, 0)),
        ],
        out_shape=[
            jax.ShapeDtypeStruct((N_TOKENS, N_Q), jnp.int32),
            jax.ShapeDtypeStruct((N_TOKENS, DIM), jnp.float32),
        ],
        scratch_shapes=[
            pltpu.VMEM((3, N_Q * BINS, DIM), jnp.bfloat16),
            pltpu.VMEM((8, N_Q * BINS), jnp.float32),
        ],
    )(hidden_states, codebooks)
    return jnp.transpose(codes_t), quant


# speedup vs baseline: 1.2825x; 1.2825x over previous
import jax
import jax.numpy as jnp
from jax import lax
from jax.experimental import pallas as pl

N_TOKENS = 1024
DIM = 256
N_Q = 4
BINS = 512


def _split3_bf16(x):
    parts = []
    r = x
    for _ in range(3):
        c = r.astype(jnp.bfloat16)
        parts.append(c)
        r = r - c.astype(jnp.float32)
    return parts


def _rvq_kernel(h_ref, cb_ref, codes_ref, quant_ref):
    residual = h_ref[:]
    ones8 = jnp.ones((8, DIM), jnp.float32)
    cb_all = cb_ref[:].reshape(N_Q * BINS, DIM)
    chalf8 = 0.5 * lax.dot_general(
        ones8, cb_all * cb_all,
        dimension_numbers=(((1,), (1,)), ((), ())),
        preferred_element_type=jnp.float32,
        precision=lax.Precision.HIGHEST,
    )
    splits = _split3_bf16(cb_all)
    idx_cols = []
    for i in range(N_Q):
        cb = cb_ref[i]
        dots = lax.dot_general(
            residual, cb,
            dimension_numbers=(((1,), (1,)), ((), ())),
            preferred_element_type=jnp.float32,
            precision=lax.Precision.HIGHEST,
        )
        scores = dots - chalf8[0:1, i * BINS:(i + 1) * BINS]
        maxs = jnp.max(scores, axis=1, keepdims=True)
        iota = lax.broadcasted_iota(jnp.int32, scores.shape, 1)
        idx2d = jnp.min(jnp.where(scores == maxs, iota, BINS),
                        axis=1, keepdims=True)
        onehot = (iota == idx2d).astype(jnp.bfloat16)
        chosen = jnp.zeros_like(residual)
        for part in splits:
            chosen = chosen + lax.dot_general(
                onehot, part[i * BINS:(i + 1) * BINS],
                dimension_numbers=(((1,), (0,)), ((), ())),
                preferred_element_type=jnp.float32,
            )
        residual = residual - chosen
        idx_cols.append(idx2d)
    codes_ref[:] = jnp.concatenate(idx_cols, axis=1)
    quant_ref[:] = h_ref[:] - residual


def kernel(hidden_states, codebooks):
    codes_t, quant = pl.pallas_call(
        _rvq_kernel,
        out_shape=[
            jax.ShapeDtypeStruct((N_TOKENS, N_Q), jnp.int32),
            jax.ShapeDtypeStruct((N_TOKENS, DIM), jnp.float32),
        ],
    )(hidden_states, codebooks)
    return jnp.transpose(codes_t), quant
